# baseline (speedup 1.0000x reference)
import math

import jax
import jax.numpy as jnp
import numpy as np
from jax import lax
from jax.experimental import pallas as pl
from jax.experimental.pallas import tpu as pltpu
from jax.experimental.pallas import tpu_sc as plsc

B = 16384
L = 10
D = 64
DP = 128
NC = 2
NS = 16
NW = NC * NS
PPW = B // NW
CH = 16
NCHUNK = PPW // CH
NR = L - 1
REL_PAD_ROW = 256


def _pos_enc() -> np.ndarray:
    pe = np.zeros((L, D), dtype=np.float32)
    position = np.arange(0, L, dtype=np.float32)[:, None]
    div_term = np.exp(np.arange(0, D, 2).astype(np.float32) * (-math.log(10000.0) / D))
    pe[:, 0::2] = np.sin(position * div_term)
    pe[:, 1::2] = np.cos(position * div_term)
    return pe


_NIDX = CH * L
_RIDX = CH * NR


def _idx_slices(total):
    out = []
    off = 0
    while off < total:
        n = min(128, total - off)
        out.append((off, n))
        off += n
    return out


def _sc_body(paths_hbm, rels_hbm, pshift_hbm, node_hbm, relpad_hbm, out_hbm,
             pvw, rvw, svw, nra, nrb, rra, rrb, oba, obb,
             sga, sgb, soa, sob):
    nrows2 = (nra, nrb)
    rrows2 = (rra, rrb)
    obuf2 = (oba, obb)
    semg = (sga, sgb)
    semo = (soa, sob)
    c = lax.axis_index("c")
    s = lax.axis_index("s")
    wid = s * NC + c
    wbase = wid * PPW

    pltpu.sync_copy(paths_hbm.at[pl.ds(wbase * L, PPW * L)], pvw)
    pltpu.sync_copy(rels_hbm.at[pl.ds(wbase * NR, PPW * NR)], rvw)
    pltpu.sync_copy(pshift_hbm.at[pl.ds(wbase * NR, PPW * NR)], svw)

    def redirect(i, carry2):
        sl = pl.ds(i * 16, 16)
        rvw[sl] = jnp.where(svw[sl] != 0, rvw[sl], REL_PAD_ROW)
        return carry2

    lax.fori_loop(0, PPW * NR // 16, redirect, 0)

    def stage(k, b):
        for off, n in _idx_slices(_NIDX):
            ioff = pl.multiple_of(k * _NIDX + off, 8)
            pltpu.async_copy(node_hbm.at[pvw.at[pl.ds(ioff, n)]],
                             nrows2[b].at[pl.ds(off, n)], semg[b])
        for off, n in _idx_slices(_RIDX):
            ioff = pl.multiple_of(k * _RIDX + off, 8)
            pltpu.async_copy(relpad_hbm.at[rvw.at[pl.ds(ioff, n)]],
                             rrows2[b].at[pl.ds(off, n)], semg[b])

    def wait_gathers(k, b):
        for off, n in _idx_slices(_NIDX):
            ioff = pl.multiple_of(k * _NIDX + off, 8)
            pltpu.make_async_copy(node_hbm.at[pvw.at[pl.ds(ioff, n)]],
                                  nrows2[b].at[pl.ds(off, n)], semg[b]).wait()
        for off, n in _idx_slices(_RIDX):
            ioff = pl.multiple_of(k * _RIDX + off, 8)
            pltpu.make_async_copy(relpad_hbm.at[rvw.at[pl.ds(ioff, n)]],
                                  rrows2[b].at[pl.ds(off, n)], semg[b]).wait()

    def out_rows(k):
        return pl.ds(pl.multiple_of(wbase + k * CH, CH), CH)

    def acc_and_flush(k, b):
        nrows, rrows, obuf = nrows2[b], rrows2[b], obuf2[b]

        def acc_path(p, carry2):
            bn = p * L
            br = p * NR
            for q in range(D // 16):
                sl = pl.ds(q * 16, 16)
                a = nrows[bn, sl]
                for l in range(1, L):
                    a = a + nrows[bn + l, sl]
                for j in range(NR):
                    a = a + rrows[br + j, sl]
                obuf[p, sl] = a
            return carry2

        lax.fori_loop(0, CH, acc_path, 0)
        pltpu.async_copy(obuf, out_hbm.at[out_rows(k)], semo[b])

    def wait_flush(k, b):
        pltpu.make_async_copy(obuf2[b], out_hbm.at[out_rows(k)], semo[b]).wait()

    stage(0, 0)

    def pairbody(i, carry):
        k0 = i * 2
        k1 = k0 + 1
        stage(k1, 1)
        wait_gathers(k0, 0)

        @pl.when(i > 0)
        def _():
            wait_flush(k0 - 2, 0)

        acc_and_flush(k0, 0)

        @pl.when(i < NCHUNK // 2 - 1)
        def _():
            stage(k0 + 2, 0)

        wait_gathers(k1, 1)

        @pl.when(i > 0)
        def _():
            wait_flush(k1 - 2, 1)

        acc_and_flush(k1, 1)
        return carry

    lax.fori_loop(0, NCHUNK // 2, pairbody, 0)
    wait_flush(NCHUNK - 2, 0)
    wait_flush(NCHUNK - 1, 1)


@jax.jit
def _sc_sums(paths_f, rels_f, pshift_f, node128, relpad128):
    mesh = plsc.VectorSubcoreMesh(core_axis_name="c", subcore_axis_name="s")
    f = pl.kernel(
        _sc_body,
        out_type=jax.ShapeDtypeStruct((B, D), jnp.float32),
        mesh=mesh,
        scratch_types=(
            [pltpu.VMEM((PPW * L,), jnp.int32)]
            + [pltpu.VMEM((PPW * NR,), jnp.int32)] * 2
            + [pltpu.VMEM((_NIDX, DP), jnp.float32)] * 2
            + [pltpu.VMEM((_RIDX, DP), jnp.float32)] * 2
            + [pltpu.VMEM((CH, D), jnp.float32)] * 2
            + [pltpu.SemaphoreType.DMA] * 4
        ),
        compiler_params=pltpu.CompilerParams(use_tc_tiling_on_sc=True),
    )
    return f(paths_f, rels_f, pshift_f, node128, relpad128)


def _tc_body(sums_ref, paths_ref, row0_ref, pe_ref, w1_ref, b1_ref,
             w2_ref, b2_ref, out_ref):
    maskf = (paths_ref[...] != 0).astype(jnp.float32)
    dsum = jnp.sum(maskf, axis=1, keepdims=True)
    denom = jnp.maximum(dsum, 1.0)
    cnt0 = jnp.float32(L) - dsum
    pe_pool = jnp.dot(maskf, pe_ref[...], preferred_element_type=jnp.float32)
    pooled = (sums_ref[...] + pe_pool - cnt0 * row0_ref[...]) / denom
    h = jnp.maximum(
        jnp.dot(pooled, w1_ref[...], preferred_element_type=jnp.float32)
        + b1_ref[...], 0.0)
    out_ref[...] = (
        jnp.dot(h, w2_ref[...], preferred_element_type=jnp.float32)
        + b2_ref[...])


@jax.jit
def _tc_mlp(sums, paths_pad, row0, pe_pad, W1, b1, W2, b2):
    blk = 512
    grid = B // blk
    return pl.pallas_call(
        _tc_body,
        grid=(grid,),
        in_specs=[
            pl.BlockSpec((blk, D), lambda i: (i, 0)),
            pl.BlockSpec((blk, 16), lambda i: (i, 0)),
            pl.BlockSpec((1, D), lambda i: (0, 0)),
            pl.BlockSpec((16, D), lambda i: (0, 0)),
            pl.BlockSpec((D, D), lambda i: (0, 0)),
            pl.BlockSpec((1, D), lambda i: (0, 0)),
            pl.BlockSpec((D, D), lambda i: (0, 0)),
            pl.BlockSpec((1, D), lambda i: (0, 0)),
        ],
        out_specs=pl.BlockSpec((blk, D), lambda i: (i, 0)),
        out_shape=jax.ShapeDtypeStruct((B, D), jnp.float32),
    )(sums, paths_pad, row0, pe_pad, W1, b1, W2, b2)


def kernel(paths, rels, node_table, rel_table, W1, b1, W2, b2):
    paths = paths.astype(jnp.int32)
    rels = rels.astype(jnp.int32)
    paths_f = paths.reshape(B * L)
    rels_f = rels.reshape(B * NR)
    pshift_f = paths[:, 1:].reshape(B * NR)
    node128 = jnp.pad(node_table, ((0, 0), (0, DP - D)))
    relpad128 = jnp.pad(rel_table, ((0, 8), (0, DP - D)))
    sums = _sc_sums(paths_f, rels_f, pshift_f, node128, relpad128)

    paths_pad = jnp.concatenate(
        [paths, jnp.zeros((B, 16 - L), dtype=jnp.int32)], axis=1)
    pe_pad = jnp.asarray(np.pad(_pos_enc(), ((0, 16 - L), (0, 0))))
    return _tc_mlp(sums, paths_pad, node_table[0:1], pe_pad,
                   W1, b1.reshape(1, D), W2, b2.reshape(1, D))


# speedup vs baseline: 1.0020x; 1.0020x over previous
import math

import jax
import jax.numpy as jnp
import numpy as np
from jax import lax
from jax.experimental import pallas as pl
from jax.experimental.pallas import tpu as pltpu
from jax.experimental.pallas import tpu_sc as plsc

B = 16384
L = 10
D = 64
DP = 128
NC = 2
NS = 16
NW = NC * NS
PPW = B // NW
CH = 16
NCHUNK = PPW // CH
NR = L - 1
REL_PAD_ROW = 256


def _pos_enc() -> np.ndarray:
    pe = np.zeros((L, D), dtype=np.float32)
    position = np.arange(0, L, dtype=np.float32)[:, None]
    div_term = np.exp(np.arange(0, D, 2).astype(np.float32) * (-math.log(10000.0) / D))
    pe[:, 0::2] = np.sin(position * div_term)
    pe[:, 1::2] = np.cos(position * div_term)
    return pe


_NIDX = CH * L
_RIDX = CH * NR


def _idx_slices(total):
    out = []
    off = 0
    while off < total:
        n = min(128, total - off)
        out.append((off, n))
        off += n
    return out


def _sc_body(paths_hbm, rels_hbm, pshift_hbm, node_hbm, relpad_hbm, out_hbm,
             pvw, rvw, svw, nra, nrb, rra, rrb, oba, obb,
             sga, sgb, soa, sob):
    nrows2 = (nra, nrb)
    rrows2 = (rra, rrb)
    obuf2 = (oba, obb)
    semg = (sga, sgb)
    semo = (soa, sob)
    c = lax.axis_index("c")
    s = lax.axis_index("s")
    wid = s * NC + c
    wbase = wid * PPW

    pltpu.sync_copy(paths_hbm.at[pl.ds(wbase * L, PPW * L)], pvw)
    pltpu.sync_copy(rels_hbm.at[pl.ds(wbase * NR, PPW * NR)], rvw)
    pltpu.sync_copy(pshift_hbm.at[pl.ds(wbase * NR, PPW * NR)], svw)

    def redirect(i, carry2):
        sl = pl.ds(i * 16, 16)
        rvw[sl] = jnp.where(svw[sl] != 0, rvw[sl], REL_PAD_ROW)
        return carry2

    lax.fori_loop(0, PPW * NR // 16, redirect, 0)

    def stage(k, b):
        for off, n in _idx_slices(_NIDX):
            ioff = pl.multiple_of(k * _NIDX + off, 8)
            pltpu.async_copy(node_hbm.at[pvw.at[pl.ds(ioff, n)]],
                             nrows2[b].at[pl.ds(off, n)], semg[b])
        for off, n in _idx_slices(_RIDX):
            ioff = pl.multiple_of(k * _RIDX + off, 8)
            pltpu.async_copy(relpad_hbm.at[rvw.at[pl.ds(ioff, n)]],
                             rrows2[b].at[pl.ds(off, n)], semg[b])

    def wait_gathers(k, b):
        for off, n in _idx_slices(_NIDX):
            ioff = pl.multiple_of(k * _NIDX + off, 8)
            pltpu.make_async_copy(node_hbm.at[pvw.at[pl.ds(ioff, n)]],
                                  nrows2[b].at[pl.ds(off, n)], semg[b]).wait()
        for off, n in _idx_slices(_RIDX):
            ioff = pl.multiple_of(k * _RIDX + off, 8)
            pltpu.make_async_copy(relpad_hbm.at[rvw.at[pl.ds(ioff, n)]],
                                  rrows2[b].at[pl.ds(off, n)], semg[b]).wait()

    def out_rows(k):
        return pl.ds(pl.multiple_of(wbase + k * CH, CH), CH)

    def acc_and_flush(k, b):
        nrows, rrows, obuf = nrows2[b], rrows2[b], obuf2[b]

        def acc_path(p, carry2):
            bn = p * L
            br = p * NR
            for q in range(D // 16):
                sl = pl.ds(q * 16, 16)
                a = nrows[bn, sl]
                for l in range(1, L):
                    a = a + nrows[bn + l, sl]
                for j in range(NR):
                    a = a + rrows[br + j, sl]
                obuf[p, sl] = a
            return carry2

        lax.fori_loop(0, CH, acc_path, 0)
        pltpu.async_copy(obuf, out_hbm.at[out_rows(k)], semo[b])

    def wait_flush(k, b):
        pltpu.make_async_copy(obuf2[b], out_hbm.at[out_rows(k)], semo[b]).wait()

    stage(0, 0)

    def pairbody(i, carry):
        k0 = i * 2
        k1 = k0 + 1
        stage(k1, 1)
        wait_gathers(k0, 0)

        @pl.when(i > 0)
        def _():
            wait_flush(k0 - 2, 0)

        acc_and_flush(k0, 0)

        @pl.when(i < NCHUNK // 2 - 1)
        def _():
            stage(k0 + 2, 0)

        wait_gathers(k1, 1)

        @pl.when(i > 0)
        def _():
            wait_flush(k1 - 2, 1)

        acc_and_flush(k1, 1)
        return carry

    lax.fori_loop(0, NCHUNK // 2, pairbody, 0)
    wait_flush(NCHUNK - 2, 0)
    wait_flush(NCHUNK - 1, 1)


@jax.jit
def _sc_sums(paths_f, rels_f, pshift_f, node128, relpad128):
    mesh = plsc.VectorSubcoreMesh(core_axis_name="c", subcore_axis_name="s")
    f = pl.kernel(
        _sc_body,
        out_type=jax.ShapeDtypeStruct((B, D), jnp.float32),
        mesh=mesh,
        scratch_types=(
            [pltpu.VMEM((PPW * L,), jnp.int32)]
            + [pltpu.VMEM((PPW * NR,), jnp.int32)] * 2
            + [pltpu.VMEM((_NIDX, DP), jnp.float32)] * 2
            + [pltpu.VMEM((_RIDX, DP), jnp.float32)] * 2
            + [pltpu.VMEM((CH, D), jnp.float32)] * 2
            + [pltpu.SemaphoreType.DMA] * 4
        ),
        compiler_params=pltpu.CompilerParams(use_tc_tiling_on_sc=True),
    )
    return f(paths_f, rels_f, pshift_f, node128, relpad128)


def _tc_body(sums_ref, paths_ref, row0_ref, pe_ref, w1_ref, b1_ref,
             w2_ref, b2_ref, out_ref):
    maskf = (paths_ref[...] != 0).astype(jnp.float32)
    dsum = jnp.sum(maskf, axis=1, keepdims=True)
    denom = jnp.maximum(dsum, 1.0)
    cnt0 = jnp.float32(L) - dsum
    pe_pool = jnp.dot(maskf, pe_ref[...], preferred_element_type=jnp.float32)
    pooled = (sums_ref[...] + pe_pool - cnt0 * row0_ref[...]) / denom
    h = jnp.maximum(
        jnp.dot(pooled, w1_ref[...], preferred_element_type=jnp.float32)
        + b1_ref[...], 0.0)
    out_ref[...] = (
        jnp.dot(h, w2_ref[...], preferred_element_type=jnp.float32)
        + b2_ref[...])


@jax.jit
def _tc_mlp(sums, paths_pad, row0, pe_pad, W1, b1, W2, b2):
    blk = 512
    grid = B // blk
    return pl.pallas_call(
        _tc_body,
        grid=(grid,),
        in_specs=[
            pl.BlockSpec((blk, D), lambda i: (i, 0)),
            pl.BlockSpec((blk, 16), lambda i: (i, 0)),
            pl.BlockSpec((1, D), lambda i: (0, 0)),
            pl.BlockSpec((16, D), lambda i: (0, 0)),
            pl.BlockSpec((D, D), lambda i: (0, 0)),
            pl.BlockSpec((1, D), lambda i: (0, 0)),
            pl.BlockSpec((D, D), lambda i: (0, 0)),
            pl.BlockSpec((1, D), lambda i: (0, 0)),
        ],
        out_specs=pl.BlockSpec((blk, D), lambda i: (i, 0)),
        out_shape=jax.ShapeDtypeStruct((B, D), jnp.float32),
    )(sums, paths_pad, row0, pe_pad, W1, b1, W2, b2)


def kernel(paths, rels, node_table, rel_table, W1, b1, W2, b2):
    paths = paths.astype(jnp.int32)
    rels = rels.astype(jnp.int32)
    paths_f = paths.reshape(B * L)
    rels_f = rels.reshape(B * NR)
    pshift_f = paths[:, 1:].reshape(B * NR)
    node128 = jnp.concatenate(
        [node_table, jnp.zeros((node_table.shape[0], DP - D),
                               dtype=node_table.dtype)], axis=1)
    relpad128 = jnp.pad(rel_table, ((0, 8), (0, DP - D)))
    sums = _sc_sums(paths_f, rels_f, pshift_f, node128, relpad128)

    paths_pad = jnp.concatenate(
        [paths, jnp.zeros((B, 16 - L), dtype=jnp.int32)], axis=1)
    pe_pad = jnp.asarray(np.pad(_pos_enc(), ((0, 16 - L), (0, 0))))
    return _tc_mlp(sums, paths_pad, node_table[0:1], pe_pad,
                   W1, b1.reshape(1, D), W2, b2.reshape(1, D))
